# baseline (speedup 1.0000x reference)
import jax
import jax.numpy as jnp
from jax.experimental import pallas as pl
from jax.experimental.pallas import tpu as pltpu

BB = 128
AUXD = 8 + 3 * 4


def _fused_kernel(x_ref, aux_ref,
                  w1_ref, sc1_ref, sh1_ref, w2_ref, sc2_ref, sh2_ref,
                  wln_ref, bln_ref,
                  wc1a_ref, wc1y_ref, bc1_ref, wc2_ref, bc2_ref,
                  wa1a_ref, wa1y_ref, ba1_ref, wa2_ref, ba2_ref,
                  value_ref, policy_ref,
                  xw_ref, a1w_ref,
                  r1a_ref, r1b_ref, r1c_ref, r1d_ref,
                  r2a_ref, r2b_ref, r2c_ref, r2d_ref):
    f32 = jnp.float32
    bb = x_ref.shape[-1]
    r1_refs = [r1a_ref, r1b_ref, r1c_ref, r1d_ref]
    r2_refs = [r2a_ref, r2b_ref, r2c_ref, r2d_ref]

    w1s = [[w1_ref[(co * 3 + dh) * 3 + dw] * sc1_ref[co]
            for dh in range(3) for dw in range(3)] for co in range(4)]
    sh1s = [sh1_ref[co] for co in range(4)]
    w2s = [[w2_ref[((co * 4 + ci) * 3 + dh) * 3 + dw] * sc2_ref[co]
            for ci in range(4) for dh in range(3) for dw in range(3)]
           for co in range(4)]
    sh2s = [sh2_ref[co] for co in range(4)]

    x = x_ref[...]
    zrow = jnp.zeros((3, 1, 64, bb), f32)
    xw_ref[:, 0:1, :, :] = zrow
    xw_ref[:, 65:66, :, :] = zrow
    zcol = jnp.zeros((64, 1, bb), f32)
    xw_ref[1, 1:65, :, :] = x
    xw_ref[0, 1:65, 1:64, :] = x[:, 0:63, :]
    xw_ref[0, 1:65, 0:1, :] = zcol
    xw_ref[2, 1:65, 0:63, :] = x[:, 1:64, :]
    xw_ref[2, 1:65, 63:64, :] = zcol

    def body1(s, carry):
        h0 = s * 2
        for co in range(4):
            acc = None
            for dh in range(3):
                for dw in range(3):
                    term = w1s[co][dh * 3 + dw] * xw_ref[dw, pl.ds(h0 + dh, 2), :, :]
                    acc = term if acc is None else acc + term
            y = jnp.maximum(acc + sh1s[co], 0.0)
            r1_refs[co][pl.ds(s, 1), :, :] = jnp.max(y, axis=0, keepdims=True)
        return carry
    jax.lax.fori_loop(0, 32, body1, 0, unroll=16)

    zrow2 = jnp.zeros((3, 4, 1, 32, bb), f32)
    a1w_ref[:, :, 0:1, :, :] = zrow2
    a1w_ref[:, :, 33:34, :, :] = zrow2
    zcol2 = jnp.zeros((32, 1, bb), f32)
    for ci in range(4):
        p = jnp.maximum(r1_refs[ci][:, 0::2, :], r1_refs[ci][:, 1::2, :])
        a1w_ref[1, ci, 1:33, :, :] = p
        a1w_ref[0, ci, 1:33, 1:32, :] = p[:, 0:31, :]
        a1w_ref[0, ci, 1:33, 0:1, :] = zcol2
        a1w_ref[2, ci, 1:33, 0:31, :] = p[:, 1:32, :]
        a1w_ref[2, ci, 1:33, 31:32, :] = zcol2

    def body2(s, carry):
        h0 = s * 2
        accs = [None] * 4
        for k in range(36):
            ci, t = divmod(k, 9)
            dh, dw = divmod(t, 3)
            win = a1w_ref[dw, ci, pl.ds(h0 + dh, 2), :, :]
            for co in range(4):
                term = w2s[co][k] * win
                accs[co] = term if accs[co] is None else accs[co] + term
        for co in range(4):
            y = jnp.maximum(accs[co] + sh2s[co], 0.0)
            r2_refs[co][pl.ds(s, 1), :, :] = jnp.max(y, axis=0, keepdims=True)
        return carry
    jax.lax.fori_loop(0, 16, body2, 0, unroll=16)

    hparts = []
    for co in range(4):
        p = jnp.maximum(r2_refs[co][:, 0::2, :], r2_refs[co][:, 1::2, :])
        hparts.append(p.reshape(256, bb))
    h = jnp.concatenate(hparts, axis=0)

    y32 = jnp.dot(wln_ref[...], h, preferred_element_type=f32) + bln_ref[...]
    aux = aux_ref[...]
    hc = jnp.maximum(
        jnp.dot(wc1a_ref[...], aux, preferred_element_type=f32)
        + jnp.dot(wc1y_ref[...], y32, preferred_element_type=f32)
        + bc1_ref[...], 0.0)
    value_ref[...] = (jnp.dot(wc2_ref[...], hc, preferred_element_type=f32)
                      + bc2_ref[...])
    ha = jnp.maximum(
        jnp.dot(wa1a_ref[...], aux, preferred_element_type=f32)
        + jnp.dot(wa1y_ref[...], y32, preferred_element_type=f32)
        + ba1_ref[...], 0.0)
    logits = (jnp.dot(wa2_ref[...], ha, preferred_element_type=f32)
              + ba2_ref[...])
    m = jnp.max(logits, axis=0, keepdims=True)
    e = jnp.exp(logits - m)
    policy_ref[...] = e / jnp.sum(e, axis=0, keepdims=True)


def kernel(sc1, sh1, sc2, sh2, w1, w2, wln, bln, wc1_aux, wc1_y, bc1, wc2, bc2,
           wa1_aux, wa1_y, ba1, wa2, ba2, x0, x1, x2):
    b = x2.shape[0]
    na = wa2.shape[1]
    xt = x2.reshape(b, 64 * 64).T.reshape(64, 64, b)
    aux_t = jnp.concatenate([x0.reshape(b, -1), x1.reshape(b, -1)], axis=1).T

    smem = pl.BlockSpec(memory_space=pltpu.MemorySpace.SMEM)
    vmem = pl.BlockSpec(memory_space=pltpu.MemorySpace.VMEM)
    value_t, policy_t = pl.pallas_call(
        _fused_kernel,
        out_shape=(jax.ShapeDtypeStruct((1, b), jnp.float32),
                   jax.ShapeDtypeStruct((na, b), jnp.float32)),
        grid=(b // BB,),
        in_specs=[
            pl.BlockSpec((64, 64, BB), lambda i: (0, 0, i)),
            pl.BlockSpec((AUXD, BB), lambda i: (0, i)),
            smem, smem, smem, smem, smem, smem,
            vmem, vmem, vmem, vmem, vmem, vmem, vmem,
            vmem, vmem, vmem, vmem, vmem,
        ],
        out_specs=(pl.BlockSpec((1, BB), lambda i: (0, i)),
                   pl.BlockSpec((na, BB), lambda i: (0, i))),
        scratch_shapes=[pltpu.VMEM((3, 66, 64, BB), jnp.float32),
                        pltpu.VMEM((3, 4, 34, 32, BB), jnp.float32)]
                       + [pltpu.VMEM((32, 64, BB), jnp.float32)] * 4
                       + [pltpu.VMEM((16, 32, BB), jnp.float32)] * 4,
        compiler_params=pltpu.CompilerParams(dimension_semantics=("parallel",)),
    )(xt, aux_t, w1, sc1, sh1, w2, sc2, sh2,
      wln.T, bln.reshape(-1, 1),
      wc1_aux.T, wc1_y.T, bc1.reshape(-1, 1), wc2, bc2,
      wa1_aux.T, wa1_y.T, ba1.reshape(-1, 1), wa2.T, ba2.reshape(-1, 1))
    return value_t.T, policy_t.T


# speedup vs baseline: 1.0380x; 1.0380x over previous
import jax
import jax.numpy as jnp
from jax.experimental import pallas as pl
from jax.experimental.pallas import tpu as pltpu

BB = 128
AUXD = 8 + 3 * 4


def _fused_kernel(x_ref, aux_ref,
                  w1_ref, sc1_ref, sh1_ref, w2_ref, sc2_ref, sh2_ref,
                  wln_ref, bln_ref,
                  wc1a_ref, wc1y_ref, bc1_ref, wc2_ref, bc2_ref,
                  wa1a_ref, wa1y_ref, ba1_ref, wa2_ref, ba2_ref,
                  value_ref, policy_ref,
                  xw_ref, a1w_ref,
                  r1a_ref, r1b_ref, r1c_ref, r1d_ref,
                  r2a_ref, r2b_ref, r2c_ref, r2d_ref):
    f32 = jnp.float32
    bb = x_ref.shape[-1]
    r1_refs = [r1a_ref, r1b_ref, r1c_ref, r1d_ref]
    r2_refs = [r2a_ref, r2b_ref, r2c_ref, r2d_ref]

    w1s = [[w1_ref[(co * 3 + dh) * 3 + dw] * sc1_ref[co]
            for dh in range(3) for dw in range(3)] for co in range(4)]
    sh1s = [sh1_ref[co] for co in range(4)]
    w2s = [[w2_ref[((co * 4 + ci) * 3 + dh) * 3 + dw] * sc2_ref[co]
            for ci in range(4) for dh in range(3) for dw in range(3)]
           for co in range(4)]
    sh2s = [sh2_ref[co] for co in range(4)]

    zrow = jnp.zeros((1, 66, bb), f32)
    xw_ref[0:1, :, :] = zrow
    xw_ref[65:66, :, :] = zrow
    zcol = jnp.zeros((64, 1, bb), f32)
    xw_ref[1:65, 0:1, :] = zcol
    xw_ref[1:65, 65:66, :] = zcol
    xw_ref[1:65, 1:65, :] = x_ref[...]

    def body1(s, carry):
        h0 = s * 2
        for co in range(4):
            acc = None
            for dh in range(3):
                for dw in range(3):
                    term = w1s[co][dh * 3 + dw] * xw_ref[pl.ds(h0 + dh, 2), dw:dw + 64, :]
                    acc = term if acc is None else acc + term
            y = jnp.maximum(acc + sh1s[co], 0.0)
            r1_refs[co][pl.ds(s, 1), :, :] = jnp.max(y, axis=0, keepdims=True)
        return carry
    jax.lax.fori_loop(0, 32, body1, 0, unroll=16)

    zrow2 = jnp.zeros((4, 1, 34, bb), f32)
    a1w_ref[:, 0:1, :, :] = zrow2
    a1w_ref[:, 33:34, :, :] = zrow2
    zcol2 = jnp.zeros((32, 1, bb), f32)
    for ci in range(4):
        p = jnp.maximum(r1_refs[ci][:, 0::2, :], r1_refs[ci][:, 1::2, :])
        a1w_ref[ci, 1:33, 1:33, :] = p
        a1w_ref[ci, 1:33, 0:1, :] = zcol2
        a1w_ref[ci, 1:33, 33:34, :] = zcol2

    def body2(s, carry):
        h0 = s * 2
        for co in range(4):
            acc = None
            for k in range(36):
                ci, t = divmod(k, 9)
                dh, dw = divmod(t, 3)
                term = w2s[co][k] * a1w_ref[ci, pl.ds(h0 + dh, 2), dw:dw + 32, :]
                acc = term if acc is None else acc + term
            y = jnp.maximum(acc + sh2s[co], 0.0)
            r2_refs[co][pl.ds(s, 1), :, :] = jnp.max(y, axis=0, keepdims=True)
        return carry
    jax.lax.fori_loop(0, 16, body2, 0, unroll=16)

    hparts = []
    for co in range(4):
        p = jnp.maximum(r2_refs[co][:, 0::2, :], r2_refs[co][:, 1::2, :])
        hparts.append(p.reshape(256, bb))
    h = jnp.concatenate(hparts, axis=0)

    y32 = jnp.dot(wln_ref[...], h, preferred_element_type=f32) + bln_ref[...]
    aux = aux_ref[...]
    hc = jnp.maximum(
        jnp.dot(wc1a_ref[...], aux, preferred_element_type=f32)
        + jnp.dot(wc1y_ref[...], y32, preferred_element_type=f32)
        + bc1_ref[...], 0.0)
    value_ref[...] = (jnp.dot(wc2_ref[...], hc, preferred_element_type=f32)
                      + bc2_ref[...])
    ha = jnp.maximum(
        jnp.dot(wa1a_ref[...], aux, preferred_element_type=f32)
        + jnp.dot(wa1y_ref[...], y32, preferred_element_type=f32)
        + ba1_ref[...], 0.0)
    logits = (jnp.dot(wa2_ref[...], ha, preferred_element_type=f32)
              + ba2_ref[...])
    m = jnp.max(logits, axis=0, keepdims=True)
    e = jnp.exp(logits - m)
    policy_ref[...] = e / jnp.sum(e, axis=0, keepdims=True)


def kernel(sc1, sh1, sc2, sh2, w1, w2, wln, bln, wc1_aux, wc1_y, bc1, wc2, bc2,
           wa1_aux, wa1_y, ba1, wa2, ba2, x0, x1, x2):
    b = x2.shape[0]
    na = wa2.shape[1]
    xt = x2.reshape(b, 64 * 64).T.reshape(64, 64, b)
    aux_t = jnp.concatenate([x0.reshape(b, -1), x1.reshape(b, -1)], axis=1).T

    smem = pl.BlockSpec(memory_space=pltpu.MemorySpace.SMEM)
    vmem = pl.BlockSpec(memory_space=pltpu.MemorySpace.VMEM)
    value_t, policy_t = pl.pallas_call(
        _fused_kernel,
        out_shape=(jax.ShapeDtypeStruct((1, b), jnp.float32),
                   jax.ShapeDtypeStruct((na, b), jnp.float32)),
        grid=(b // BB,),
        in_specs=[
            pl.BlockSpec((64, 64, BB), lambda i: (0, 0, i)),
            pl.BlockSpec((AUXD, BB), lambda i: (0, i)),
            smem, smem, smem, smem, smem, smem,
            vmem, vmem, vmem, vmem, vmem, vmem, vmem,
            vmem, vmem, vmem, vmem, vmem,
        ],
        out_specs=(pl.BlockSpec((1, BB), lambda i: (0, i)),
                   pl.BlockSpec((na, BB), lambda i: (0, i))),
        scratch_shapes=[pltpu.VMEM((66, 66, BB), jnp.float32),
                        pltpu.VMEM((4, 34, 34, BB), jnp.float32)]
                       + [pltpu.VMEM((32, 64, BB), jnp.float32)] * 4
                       + [pltpu.VMEM((16, 32, BB), jnp.float32)] * 4,
        compiler_params=pltpu.CompilerParams(dimension_semantics=("parallel",)),
    )(xt, aux_t, w1, sc1, sh1, w2, sc2, sh2,
      wln.T, bln.reshape(-1, 1),
      wc1_aux.T, wc1_y.T, bc1.reshape(-1, 1), wc2, bc2,
      wa1_aux.T, wa1_y.T, ba1.reshape(-1, 1), wa2.T, ba2.reshape(-1, 1))
    return value_t.T, policy_t.T


# speedup vs baseline: 1.0392x; 1.0011x over previous
import jax
import jax.numpy as jnp
from jax.experimental import pallas as pl
from jax.experimental.pallas import tpu as pltpu

BB = 128
AUXD = 8 + 3 * 4


def _fused_kernel(x_ref, aux_ref,
                  w1_ref, sc1_ref, sh1_ref, w2_ref, sc2_ref, sh2_ref,
                  wln_ref, bln_ref,
                  wc1a_ref, wc1y_ref, bc1_ref, wc2_ref, bc2_ref,
                  wa1a_ref, wa1y_ref, ba1_ref, wa2_ref, ba2_ref,
                  value_ref, policy_ref,
                  xp_ref, a1p_ref,
                  r1a_ref, r1b_ref, r1c_ref, r1d_ref,
                  r2a_ref, r2b_ref, r2c_ref, r2d_ref):
    f32 = jnp.float32
    bb = x_ref.shape[-1]
    r1_refs = [r1a_ref, r1b_ref, r1c_ref, r1d_ref]
    r2_refs = [r2a_ref, r2b_ref, r2c_ref, r2d_ref]

    w1s = [[w1_ref[(co * 3 + dh) * 3 + dw] * sc1_ref[co]
            for dh in range(3) for dw in range(3)] for co in range(4)]
    sh1s = [sh1_ref[co] for co in range(4)]
    w2s = [[w2_ref[((co * 4 + ci) * 3 + dh) * 3 + dw] * sc2_ref[co]
            for ci in range(4) for dh in range(3) for dw in range(3)]
           for co in range(4)]
    sh2s = [sh2_ref[co] for co in range(4)]

    zrow = jnp.zeros((1, 66, bb), f32)
    xp_ref[0:1, :, :] = zrow
    xp_ref[65:66, :, :] = zrow
    zcol = jnp.zeros((64, 1, bb), f32)
    xp_ref[1:65, 0:1, :] = zcol
    xp_ref[1:65, 65:66, :] = zcol
    xp_ref[1:65, 1:65, :] = x_ref[...]

    def body1(s, carry):
        h0 = s * 2
        for co in range(4):
            acc = None
            for dh in range(3):
                for dw in range(3):
                    term = w1s[co][dh * 3 + dw] * xp_ref[pl.ds(h0 + dh, 2), dw:dw + 64, :]
                    acc = term if acc is None else acc + term
            y = jnp.maximum(acc + sh1s[co], 0.0)
            r1_refs[co][pl.ds(s, 1), :, :] = jnp.max(y, axis=0, keepdims=True)
        return carry
    jax.lax.fori_loop(0, 32, body1, 0, unroll=16)

    zrow2 = jnp.zeros((4, 1, 34, bb), f32)
    a1p_ref[:, 0:1, :, :] = zrow2
    a1p_ref[:, 33:34, :, :] = zrow2
    zcol2 = jnp.zeros((32, 1, bb), f32)
    for ci in range(4):
        p = jnp.maximum(r1_refs[ci][:, 0::2, :], r1_refs[ci][:, 1::2, :])
        a1p_ref[ci, 1:33, 1:33, :] = p
        a1p_ref[ci, 1:33, 0:1, :] = zcol2
        a1p_ref[ci, 1:33, 33:34, :] = zcol2

    def body2(s, carry):
        h0 = s * 2
        for co in range(4):
            acc = None
            for k in range(36):
                ci, t = divmod(k, 9)
                dh, dw = divmod(t, 3)
                term = w2s[co][k] * a1p_ref[ci, pl.ds(h0 + dh, 2), dw:dw + 32, :]
                acc = term if acc is None else acc + term
            y = jnp.maximum(acc + sh2s[co], 0.0)
            r2_refs[co][pl.ds(s, 1), :, :] = jnp.max(y, axis=0, keepdims=True)
        return carry
    jax.lax.fori_loop(0, 16, body2, 0, unroll=16)

    hparts = []
    for co in range(4):
        p = jnp.maximum(r2_refs[co][:, 0::2, :], r2_refs[co][:, 1::2, :])
        hparts.append(p.reshape(256, bb))
    h = jnp.concatenate(hparts, axis=0)

    y32 = jnp.dot(wln_ref[...], h, preferred_element_type=f32) + bln_ref[...]
    aux = aux_ref[...]
    hc = jnp.maximum(
        jnp.dot(wc1a_ref[...], aux, preferred_element_type=f32)
        + jnp.dot(wc1y_ref[...], y32, preferred_element_type=f32)
        + bc1_ref[...], 0.0)
    value_ref[...] = (jnp.dot(wc2_ref[...], hc, preferred_element_type=f32)
                      + bc2_ref[...])
    ha = jnp.maximum(
        jnp.dot(wa1a_ref[...], aux, preferred_element_type=f32)
        + jnp.dot(wa1y_ref[...], y32, preferred_element_type=f32)
        + ba1_ref[...], 0.0)
    logits = (jnp.dot(wa2_ref[...], ha, preferred_element_type=f32)
              + ba2_ref[...])
    m = jnp.max(logits, axis=0, keepdims=True)
    e = jnp.exp(logits - m)
    policy_ref[...] = e / jnp.sum(e, axis=0, keepdims=True)


def kernel(sc1, sh1, sc2, sh2, w1, w2, wln, bln, wc1_aux, wc1_y, bc1, wc2, bc2,
           wa1_aux, wa1_y, ba1, wa2, ba2, x0, x1, x2):
    b = x2.shape[0]
    na = wa2.shape[1]
    xt = x2.reshape(b, 64 * 64).T.reshape(64, 64, b)
    aux_t = jnp.concatenate([x0.reshape(b, -1), x1.reshape(b, -1)], axis=1).T

    smem = pl.BlockSpec(memory_space=pltpu.MemorySpace.SMEM)
    vmem = pl.BlockSpec(memory_space=pltpu.MemorySpace.VMEM)
    value_t, policy_t = pl.pallas_call(
        _fused_kernel,
        out_shape=(jax.ShapeDtypeStruct((1, b), jnp.float32),
                   jax.ShapeDtypeStruct((na, b), jnp.float32)),
        grid=(b // BB,),
        in_specs=[
            pl.BlockSpec((64, 64, BB), lambda i: (0, 0, i)),
            pl.BlockSpec((AUXD, BB), lambda i: (0, i)),
            smem, smem, smem, smem, smem, smem,
            vmem, vmem, vmem, vmem, vmem, vmem, vmem,
            vmem, vmem, vmem, vmem, vmem,
        ],
        out_specs=(pl.BlockSpec((1, BB), lambda i: (0, i)),
                   pl.BlockSpec((na, BB), lambda i: (0, i))),
        scratch_shapes=[pltpu.VMEM((66, 66, BB), jnp.float32),
                        pltpu.VMEM((4, 34, 34, BB), jnp.float32)]
                       + [pltpu.VMEM((32, 64, BB), jnp.float32)] * 4
                       + [pltpu.VMEM((16, 32, BB), jnp.float32)] * 4,
        compiler_params=pltpu.CompilerParams(dimension_semantics=("parallel",)),
    )(xt, aux_t, w1, sc1, sh1, w2, sc2, sh2,
      wln.T, bln.reshape(-1, 1),
      wc1_aux.T, wc1_y.T, bc1.reshape(-1, 1), wc2, bc2,
      wa1_aux.T, wa1_y.T, ba1.reshape(-1, 1), wa2.T, ba2.reshape(-1, 1))
    return value_t.T, policy_t.T
